# baseline (speedup 1.0000x reference)
import functools

import jax
import jax.numpy as jnp
from jax import lax
from jax.experimental import pallas as pl
from jax.experimental.pallas import tpu as pltpu
from jax.experimental.pallas import tpu_sc as plsc

V = 1000001
D = 32
B = 16384
NC = 2
NS = 16
NW = NC * NS
BPW = B // NW


def _glove_body(ctr_hbm, cxt_hbm,
                ctr_tab_hbm, cxt_tab_hbm, ctr_b_hbm, cxt_b_hbm,
                ctr_emb_out, cxt_emb_out, ctr_bias_out, cxt_bias_out,
                idx_ctr, idx_cxt, rows_ctr, rows_cxt, bias_ctr, bias_cxt,
                sem0, sem1, sem2, sem3):
    wid = lax.axis_index("s") * NC + lax.axis_index("c")
    base = wid * BPW

    pltpu.sync_copy(ctr_hbm.at[pl.ds(base, BPW)], idx_ctr)
    pltpu.sync_copy(cxt_hbm.at[pl.ds(base, BPW)], idx_cxt)

    c0 = pltpu.async_copy(ctr_tab_hbm.at[idx_ctr], rows_ctr, sem0)
    c1 = pltpu.async_copy(cxt_tab_hbm.at[idx_cxt], rows_cxt, sem1)
    c2 = pltpu.async_copy(ctr_b_hbm.at[idx_ctr], bias_ctr, sem2)
    c3 = pltpu.async_copy(cxt_b_hbm.at[idx_cxt], bias_cxt, sem3)
    c0.wait()
    c1.wait()
    c2.wait()
    c3.wait()

    pltpu.sync_copy(rows_ctr, ctr_emb_out.at[pl.ds(base, BPW)])
    pltpu.sync_copy(rows_cxt, cxt_emb_out.at[pl.ds(base, BPW)])
    pltpu.sync_copy(bias_ctr, ctr_bias_out.at[pl.ds(base, BPW)])
    pltpu.sync_copy(bias_cxt, cxt_bias_out.at[pl.ds(base, BPW)])


_glove_sc = pl.kernel(
    _glove_body,
    out_type=(
        jax.ShapeDtypeStruct((B, D), jnp.float32),
        jax.ShapeDtypeStruct((B, D), jnp.float32),
        jax.ShapeDtypeStruct((B,), jnp.float32),
        jax.ShapeDtypeStruct((B,), jnp.float32),
    ),
    mesh=plsc.VectorSubcoreMesh(
        core_axis_name="c", subcore_axis_name="s",
        num_cores=NC, num_subcores=NS),
    scratch_types=[
        pltpu.VMEM((BPW,), jnp.int32),
        pltpu.VMEM((BPW,), jnp.int32),
        pltpu.VMEM((BPW, D), jnp.float32),
        pltpu.VMEM((BPW, D), jnp.float32),
        pltpu.VMEM((BPW,), jnp.float32),
        pltpu.VMEM((BPW,), jnp.float32),
        pltpu.SemaphoreType.DMA,
        pltpu.SemaphoreType.DMA,
        pltpu.SemaphoreType.DMA,
        pltpu.SemaphoreType.DMA,
    ],
    compiler_params=pltpu.CompilerParams(use_tc_tiling_on_sc=False),
    name="glove_lookup_sc",
)


@jax.jit
def kernel(ctr, cxt, ctr_table, cxt_table, ctr_bias_table, cxt_bias_table):
    ctr = ctr.astype(jnp.int32)
    cxt = cxt.astype(jnp.int32)
    ce, xe, cb, xb = _glove_sc(ctr, cxt, ctr_table, cxt_table,
                               ctr_bias_table.reshape(V),
                               cxt_bias_table.reshape(V))
    return ce, xe, cb.reshape(B, 1), xb.reshape(B, 1)


# speedup vs baseline: 3.1061x; 3.1061x over previous
import functools
import jax
import jax.numpy as jnp
from jax import lax
from jax.experimental import pallas as pl
from jax.experimental.pallas import tpu as pltpu
from jax.experimental.pallas import tpu_sc as plsc

V = 1000001
D = 32
B = 16384
NC = 2
NS = 16
NW = NC * NS
BPW = B // NW
NBLK = BPW // 128
NBUF = 8


def _body(ctr_hbm, cxt_hbm, yc_hbm, yx_hbm, bc_hbm, bx_hbm,
          ec_out, ex_out, bc_out, bx_out,
          idx_c, idx_x, pages, rows, bias_c, bias_x,
          sem0, sem2, sem3):
    wid = lax.axis_index("s") * NC + lax.axis_index("c")
    base = wid * BPW

    pltpu.sync_copy(ctr_hbm.at[pl.ds(base, BPW)], idx_c)
    pltpu.sync_copy(cxt_hbm.at[pl.ds(base, BPW)], idx_x)

    c2 = pltpu.async_copy(bc_hbm.at[idx_c], bias_c, sem2)
    c3 = pltpu.async_copy(bx_hbm.at[idx_x], bias_x, sem3)

    lane = lax.iota(jnp.int32, 16)

    def scalar_at(vref, j):
        grp = vref[pl.ds((j // 16) * 16, 16)]
        return jax.lax.reduce_sum_p.bind(
            jnp.where(lane == (j % 16), grp, 0), axes=(0,))

    def run_table(tab_hbm, idx_ref, out_hbm):
        def fire(j):
            i_sc = scalar_at(idx_ref, j)
            page = pl.multiple_of((i_sc // 128) * 128, 128)
            pltpu.async_copy(tab_hbm.at[:, pl.ds(page, 128)],
                             pages.at[j % NBUF], sem0)

        for b in range(NBUF):
            fire(b)

        def step(j, _):
            pltpu.make_async_copy(tab_hbm.at[:, pl.ds(0, 128)],
                                  pages.at[j % NBUF], sem0).wait()
            i_sc = scalar_at(idx_ref, j)
            col = lax.rem(i_sc, 128)
            blk = j // 128
            lane_j = lax.rem(j, 128)
            for h in range(2):
                d_vec = lane + 16 * h
                vals = plsc.load_gather(
                    pages, [jnp.full((16,), j % NBUF, jnp.int32), d_vec,
                            jnp.full((16,), col, jnp.int32)])
                plsc.store_scatter(
                    rows, [jnp.full((16,), blk, jnp.int32), d_vec,
                           jnp.full((16,), lane_j, jnp.int32)], vals)

            @pl.when(j + NBUF < BPW)
            def _():
                fire(j + NBUF)
            return ()

        lax.fori_loop(0, BPW, step, ())
        pltpu.sync_copy(rows, out_hbm.at[pl.ds(wid * NBLK, NBLK)])

    run_table(yc_hbm, idx_c, ec_out)
    run_table(yx_hbm, idx_x, ex_out)

    c2.wait()
    c3.wait()
    pltpu.sync_copy(bias_c, bc_out.at[pl.ds(base, BPW)])
    pltpu.sync_copy(bias_x, bx_out.at[pl.ds(base, BPW)])


_sc_call = pl.kernel(
    _body,
    out_type=(
        jax.ShapeDtypeStruct((B // 128, D, 128), jnp.float32),
        jax.ShapeDtypeStruct((B // 128, D, 128), jnp.float32),
        jax.ShapeDtypeStruct((B,), jnp.float32),
        jax.ShapeDtypeStruct((B,), jnp.float32),
    ),
    mesh=plsc.VectorSubcoreMesh(
        core_axis_name="c", subcore_axis_name="s",
        num_cores=NC, num_subcores=NS),
    scratch_types=[
        pltpu.VMEM((BPW,), jnp.int32),
        pltpu.VMEM((BPW,), jnp.int32),
        pltpu.VMEM((NBUF, D, 128), jnp.float32),
        pltpu.VMEM((NBLK, D, 128), jnp.float32),
        pltpu.VMEM((BPW,), jnp.float32),
        pltpu.VMEM((BPW,), jnp.float32),
        pltpu.SemaphoreType.DMA,
        pltpu.SemaphoreType.DMA,
        pltpu.SemaphoreType.DMA,
    ],
    compiler_params=pltpu.CompilerParams(use_tc_tiling_on_sc=True,
                                         needs_layout_passes=False),
    name="glove_page_sc",
)


@jax.jit
def kernel(ctr, cxt, ctr_table, cxt_table, ctr_bias_table, cxt_bias_table):
    ctr = ctr.astype(jnp.int32)
    cxt = cxt.astype(jnp.int32)
    ec, ex, cb, xb = _sc_call(ctr, cxt, ctr_table.T, cxt_table.T,
                              ctr_bias_table.reshape(V),
                              cxt_bias_table.reshape(V))
    ec = ec.transpose(0, 2, 1).reshape(B, D)
    ex = ex.transpose(0, 2, 1).reshape(B, D)
    return ec, ex, cb.reshape(B, 1), xb.reshape(B, 1)


# speedup vs baseline: 3.1080x; 1.0006x over previous
import functools
import jax
import jax.numpy as jnp
from jax import lax
from jax.experimental import pallas as pl
from jax.experimental.pallas import tpu as pltpu
from jax.experimental.pallas import tpu_sc as plsc

V = 1000001
D = 32
B = 16384
NC = 2
NS = 16
NW = NC * NS
BPW = B // NW
NBLK = BPW // 128
NBUF = 8


def _body(ctr_hbm, cxt_hbm, yc_hbm, yx_hbm, bc_hbm, bx_hbm,
          ec_out, ex_out, bc_out, bx_out,
          idx_c, idx_x, pages, pages_x, rows, rows_x, bias_c, bias_x,
          sem0, sem1, sem2, sem3):
    wid = lax.axis_index("s") * NC + lax.axis_index("c")
    base = wid * BPW

    pltpu.sync_copy(ctr_hbm.at[pl.ds(base, BPW)], idx_c)
    pltpu.sync_copy(cxt_hbm.at[pl.ds(base, BPW)], idx_x)

    c2 = pltpu.async_copy(bc_hbm.at[idx_c], bias_c, sem2)
    c3 = pltpu.async_copy(bx_hbm.at[idx_x], bias_x, sem3)

    lane = lax.iota(jnp.int32, 16)

    def scalar_at(vref, j):
        grp = vref[pl.ds((j // 16) * 16, 16)]
        return jax.lax.reduce_sum_p.bind(
            jnp.where(lane == (j % 16), grp, 0), axes=(0,))

    def fire(tab_hbm, idx_ref, pg, sem, j):
        i_sc = scalar_at(idx_ref, j)
        page = pl.multiple_of((i_sc // 128) * 128, 128)
        pltpu.async_copy(tab_hbm.at[:, pl.ds(page, 128)],
                         pg.at[j % NBUF], sem)

    for b in range(NBUF):
        fire(yc_hbm, idx_c, pages, sem0, b)
        fire(yx_hbm, idx_x, pages_x, sem1, b)

    def extract(tab_hbm, idx_ref, pg, rw, sem, j):
        pltpu.make_async_copy(tab_hbm.at[:, pl.ds(0, 128)],
                              pg.at[j % NBUF], sem).wait()
        i_sc = scalar_at(idx_ref, j)
        col = lax.rem(i_sc, 128)
        blk = j // 128
        lane_j = lax.rem(j, 128)
        for h in range(2):
            d_vec = lane + 16 * h
            vals = plsc.load_gather(
                pg, [jnp.full((16,), j % NBUF, jnp.int32), d_vec,
                     jnp.full((16,), col, jnp.int32)])
            plsc.store_scatter(
                rw, [jnp.full((16,), blk, jnp.int32), d_vec,
                     jnp.full((16,), lane_j, jnp.int32)], vals)

    def step(j, _):
        extract(yc_hbm, idx_c, pages, rows, sem0, j)
        extract(yx_hbm, idx_x, pages_x, rows_x, sem1, j)

        @pl.when(j + NBUF < BPW)
        def _():
            fire(yc_hbm, idx_c, pages, sem0, j + NBUF)
            fire(yx_hbm, idx_x, pages_x, sem1, j + NBUF)
        return ()

    lax.fori_loop(0, BPW, step, ())
    pltpu.sync_copy(rows, ec_out.at[pl.ds(wid * NBLK, NBLK)])
    pltpu.sync_copy(rows_x, ex_out.at[pl.ds(wid * NBLK, NBLK)])

    c2.wait()
    c3.wait()
    pltpu.sync_copy(bias_c, bc_out.at[pl.ds(base, BPW)])
    pltpu.sync_copy(bias_x, bx_out.at[pl.ds(base, BPW)])


_sc_call = pl.kernel(
    _body,
    out_type=(
        jax.ShapeDtypeStruct((B // 128, D, 128), jnp.float32),
        jax.ShapeDtypeStruct((B // 128, D, 128), jnp.float32),
        jax.ShapeDtypeStruct((B,), jnp.float32),
        jax.ShapeDtypeStruct((B,), jnp.float32),
    ),
    mesh=plsc.VectorSubcoreMesh(
        core_axis_name="c", subcore_axis_name="s",
        num_cores=NC, num_subcores=NS),
    scratch_types=[
        pltpu.VMEM((BPW,), jnp.int32),
        pltpu.VMEM((BPW,), jnp.int32),
        pltpu.VMEM((NBUF, D, 128), jnp.float32),
        pltpu.VMEM((NBUF, D, 128), jnp.float32),
        pltpu.VMEM((NBLK, D, 128), jnp.float32),
        pltpu.VMEM((NBLK, D, 128), jnp.float32),
        pltpu.VMEM((BPW,), jnp.float32),
        pltpu.VMEM((BPW,), jnp.float32),
        pltpu.SemaphoreType.DMA,
        pltpu.SemaphoreType.DMA,
        pltpu.SemaphoreType.DMA,
        pltpu.SemaphoreType.DMA,
    ],
    compiler_params=pltpu.CompilerParams(use_tc_tiling_on_sc=True,
                                         needs_layout_passes=False),
    name="glove_page_sc",
)


@jax.jit
def kernel(ctr, cxt, ctr_table, cxt_table, ctr_bias_table, cxt_bias_table):
    ctr = ctr.astype(jnp.int32)
    cxt = cxt.astype(jnp.int32)
    ec, ex, cb, xb = _sc_call(ctr, cxt, ctr_table.T, cxt_table.T,
                              ctr_bias_table.reshape(V),
                              cxt_bias_table.reshape(V))
    ec = ec.transpose(0, 2, 1).reshape(B, D)
    ex = ex.transpose(0, 2, 1).reshape(B, D)
    return ec, ex, cb.reshape(B, 1), xb.reshape(B, 1)
